# Initial kernel scaffold; baseline (speedup 1.0000x reference)
import functools

import jax
import jax.numpy as jnp
from jax import lax
from jax.experimental import pallas as pl
from jax.experimental.pallas import tpu as pltpu
from jax.experimental.pallas import tpu_sc as plsc

N_NODES_ = 10000
N_EDGES_ = 320000
D_ = 128

NC = 2
NS = 16
NW = NC * NS
E_PER_TILE = N_EDGES_ // NW
CHUNK = 80
N_CHUNKS = E_PER_TILE // CHUNK
ROWS_PER_TILE = N_NODES_ // NS


def _sc_body(src_hbm, dst_hbm, emb_hbm, zeros_hbm, out_hbm,
             src_v, dst_v, rows_v, acc, sem):
  c = lax.axis_index("c")
  s = lax.axis_index("s")
  t = c * NS + s
  row_base = s * ROWS_PER_TILE

  pltpu.sync_copy(zeros_hbm.at[pl.ds(row_base, ROWS_PER_TILE)],
                  acc.at[pl.ds(row_base, ROWS_PER_TILE)])
  pltpu.sync_copy(src_hbm.at[t], src_v)
  pltpu.sync_copy(dst_hbm.at[t], dst_v)
  plsc.subcore_barrier()

  def chunk_step(j, carry):
    pltpu.async_copy(emb_hbm.at[src_v.at[j]], rows_v, sem).wait()
    pltpu.sync_copy(rows_v, acc.at[dst_v.at[j]], add=True)
    return carry

  lax.fori_loop(0, N_CHUNKS, chunk_step, 0)

  plsc.subcore_barrier()
  pltpu.sync_copy(acc.at[pl.ds(row_base, ROWS_PER_TILE)],
                  out_hbm.at[c, pl.ds(row_base, ROWS_PER_TILE)])


@functools.partial(
    pl.kernel,
    out_type=jax.ShapeDtypeStruct((NC, N_NODES_, D_), jnp.float32),
    mesh=plsc.VectorSubcoreMesh(core_axis_name="c", subcore_axis_name="s"),
    scratch_types=[
        pltpu.VMEM((N_CHUNKS, CHUNK), jnp.int32),
        pltpu.VMEM((N_CHUNKS, CHUNK), jnp.int32),
        pltpu.VMEM((CHUNK, D_), jnp.float32),
        pltpu.VMEM_SHARED((N_NODES_, D_), jnp.float32),
        pltpu.SemaphoreType.DMA,
    ],
)
def _sc_aggregate(src_hbm, dst_hbm, emb_hbm, zeros_hbm, out_hbm,
                  src_v, dst_v, rows_v, acc, sem):
  _sc_body(src_hbm, dst_hbm, emb_hbm, zeros_hbm, out_hbm,
           src_v, dst_v, rows_v, acc, sem)


def _add_body(a_ref, b_ref, o_ref):
  o_ref[...] = a_ref[...] + b_ref[...]


def _combine(p0, p1):
  blk = 1000
  return pl.pallas_call(
      _add_body,
      out_shape=jax.ShapeDtypeStruct((N_NODES_, D_), jnp.float32),
      grid=(N_NODES_ // blk,),
      in_specs=[pl.BlockSpec((blk, D_), lambda i: (i, 0)),
                pl.BlockSpec((blk, D_), lambda i: (i, 0))],
      out_specs=pl.BlockSpec((blk, D_), lambda i: (i, 0)),
  )(p0, p1)


def kernel(mode, edge_index, entity_embed):
  del mode
  src = edge_index[0].reshape(NW, N_CHUNKS, CHUNK)
  dst = edge_index[1].reshape(NW, N_CHUNKS, CHUNK)
  zeros = jnp.zeros((N_NODES_, D_), jnp.float32)
  partials = _sc_aggregate(src, dst, entity_embed, zeros)
  return _combine(partials[0], partials[1])


# speedup vs baseline: 7.4021x; 7.4021x over previous
import functools

import jax
import jax.numpy as jnp
from jax import lax
from jax.experimental import pallas as pl
from jax.experimental.pallas import tpu as pltpu
from jax.experimental.pallas import tpu_sc as plsc

N_NODES_ = 10000
N_EDGES_ = 320000
D_ = 128

NC = 2
NS = 16
NW = NC * NS
E_PER_TILE = N_EDGES_ // NW
CHUNK = 80
N_CHUNKS = E_PER_TILE // CHUNK
ROWS_PER_TILE = 624
TAIL_BASE = NS * ROWS_PER_TILE
TAIL_ROWS = N_NODES_ - TAIL_BASE


def _sc_body(src_hbm, dst_hbm, emb_hbm, zeros_hbm, out_hbm,
             src_v, dst_v, rows_v, acc, sem):
  c = lax.axis_index("c")
  s = lax.axis_index("s")
  t = c * NS + s
  row_base = s * ROWS_PER_TILE

  pltpu.sync_copy(zeros_hbm.at[pl.ds(row_base, ROWS_PER_TILE)],
                  acc.at[pl.ds(row_base, ROWS_PER_TILE)])

  @pl.when(s == 0)
  def _init_tail():
    pltpu.sync_copy(zeros_hbm.at[pl.ds(TAIL_BASE, TAIL_ROWS)],
                    acc.at[pl.ds(TAIL_BASE, TAIL_ROWS)])
  pltpu.sync_copy(src_hbm.at[t], src_v)
  pltpu.sync_copy(dst_hbm.at[t], dst_v)
  plsc.subcore_barrier()

  def chunk_step(j, carry):
    pltpu.async_copy(emb_hbm.at[src_v.at[j]], rows_v, sem).wait()
    pltpu.sync_copy(rows_v, acc.at[dst_v.at[j]], add=True)
    return carry

  lax.fori_loop(0, N_CHUNKS, chunk_step, 0)

  plsc.subcore_barrier()
  pltpu.sync_copy(acc.at[pl.ds(row_base, ROWS_PER_TILE)],
                  out_hbm.at[c, pl.ds(row_base, ROWS_PER_TILE)])

  @pl.when(s == 0)
  def _write_tail():
    pltpu.sync_copy(acc.at[pl.ds(TAIL_BASE, TAIL_ROWS)],
                    out_hbm.at[c, pl.ds(TAIL_BASE, TAIL_ROWS)])


@functools.partial(
    pl.kernel,
    out_type=jax.ShapeDtypeStruct((NC, N_NODES_, D_), jnp.float32),
    mesh=plsc.VectorSubcoreMesh(core_axis_name="c", subcore_axis_name="s"),
    scratch_types=[
        pltpu.VMEM((N_CHUNKS, CHUNK), jnp.int32),
        pltpu.VMEM((N_CHUNKS, CHUNK), jnp.int32),
        pltpu.VMEM((CHUNK, D_), jnp.float32),
        pltpu.VMEM_SHARED((N_NODES_, D_), jnp.float32),
        pltpu.SemaphoreType.DMA,
    ],
)
def _sc_aggregate(src_hbm, dst_hbm, emb_hbm, zeros_hbm, out_hbm,
                  src_v, dst_v, rows_v, acc, sem):
  _sc_body(src_hbm, dst_hbm, emb_hbm, zeros_hbm, out_hbm,
           src_v, dst_v, rows_v, acc, sem)


def _add_body(a_ref, b_ref, o_ref):
  o_ref[...] = a_ref[...] + b_ref[...]


def _combine(p0, p1):
  blk = 1000
  return pl.pallas_call(
      _add_body,
      out_shape=jax.ShapeDtypeStruct((N_NODES_, D_), jnp.float32),
      grid=(N_NODES_ // blk,),
      in_specs=[pl.BlockSpec((blk, D_), lambda i: (i, 0)),
                pl.BlockSpec((blk, D_), lambda i: (i, 0))],
      out_specs=pl.BlockSpec((blk, D_), lambda i: (i, 0)),
  )(p0, p1)


def kernel(mode, edge_index, entity_embed):
  del mode
  src = edge_index[0].reshape(NW, N_CHUNKS, CHUNK)
  dst = edge_index[1].reshape(NW, N_CHUNKS, CHUNK)
  zeros = jnp.zeros((N_NODES_, D_), jnp.float32)
  partials = _sc_aggregate(src, dst, entity_embed, zeros)
  return _combine(partials[0], partials[1])
